# baseline (speedup 1.0000x reference)
import functools

import jax
import jax.numpy as jnp
from jax.experimental import pallas as pl

D_MODEL = 4096
NUM_PATHS = 512
K = 51
K_PAD = 64
T_BLK = 256

_NEG_INF = float("-inf")


def _gate_kernel(x_ref, w_ref, idx_ref, scores_ref):
    x = x_ref[...]
    w = w_ref[...]
    logits = jax.lax.dot_general(
        x, w, (((1,), (1,)), ((), ())),
        preferred_element_type=jnp.float32,
    )

    iota_p = jax.lax.broadcasted_iota(jnp.int32, (T_BLK, NUM_PATHS), 1)
    iota_k = jax.lax.broadcasted_iota(jnp.int32, (T_BLK, K_PAD), 1)

    def body(i, carry):
        vals, out_v, out_i = carry
        m = jnp.max(vals, axis=1, keepdims=True)
        idx = jnp.min(
            jnp.where(vals == m, iota_p, NUM_PATHS), axis=1, keepdims=True
        )
        vals = jnp.where(iota_p == idx, _NEG_INF, vals)
        sel = iota_k == i
        out_v = jnp.where(sel, m, out_v)
        out_i = jnp.where(sel, idx, out_i)
        return vals, out_v, out_i

    out_v = jnp.full((T_BLK, K_PAD), _NEG_INF, dtype=jnp.float32)
    out_i = jnp.zeros((T_BLK, K_PAD), dtype=jnp.int32)
    _, out_v, out_i = jax.lax.fori_loop(
        0, K, body, (logits, out_v, out_i)
    )

    e = jnp.exp(out_v - out_v[:, 0:1])
    s = jnp.sum(e, axis=1, keepdims=True)
    scores = e / s

    idx_ref[...] = out_i[:, :K]
    scores_ref[...] = scores[:, :K]


@jax.jit
def kernel(x, W):
    n_tokens = x.shape[0]
    grid = (n_tokens // T_BLK,)
    idx, scores = pl.pallas_call(
        _gate_kernel,
        grid=grid,
        in_specs=[
            pl.BlockSpec((T_BLK, D_MODEL), lambda i: (i, 0)),
            pl.BlockSpec((NUM_PATHS, D_MODEL), lambda i: (0, 0)),
        ],
        out_specs=[
            pl.BlockSpec((T_BLK, K), lambda i: (i, 0)),
            pl.BlockSpec((T_BLK, K), lambda i: (i, 0)),
        ],
        out_shape=[
            jax.ShapeDtypeStruct((n_tokens, K), jnp.int32),
            jax.ShapeDtypeStruct((n_tokens, K), jnp.float32),
        ],
    )(x, W)
    return idx, scores


# speedup vs baseline: 2.8607x; 2.8607x over previous
import jax
import jax.numpy as jnp
from jax.experimental import pallas as pl

D_MODEL = 4096
NUM_PATHS = 512
K = 51
T_BLK = 256
V = 64


def _gt(a, ia, b, ib):
    return (a > b) | ((a == b) & (ia < ib))


def _sub_iota(shape, axis):
    return jax.lax.broadcasted_iota(jnp.int32, shape, axis)


def _cex(a, ia, b, ib, flip):
    g = _gt(b, ib, a, ia)
    sw = g ^ flip
    na = jnp.where(sw, b, a)
    nia = jnp.where(sw, ib, ia)
    nb = jnp.where(sw, a, b)
    nib = jnp.where(sw, ia, ib)
    return na, nia, nb, nib


def _step_full(k, ix, s, subflip_fn):
    t = k.shape[-1]
    no = V // (2 * s)
    kr = k.reshape(no, 2, s, 8, t)
    ir = ix.reshape(no, 2, s, 8, t)
    a, b = kr[:, 0], kr[:, 1]
    ia_, ib_ = ir[:, 0], ir[:, 1]
    flip = subflip_fn(_sub_iota(a.shape, 2))
    na, nia, nb, nib = _cex(a, ia_, b, ib_, flip)
    k = jnp.stack([na, nb], axis=1).reshape(V, 8, t)
    ix = jnp.stack([nia, nib], axis=1).reshape(V, 8, t)
    return k, ix


def _sort64(k, ix, subflip_fn):
    t = k.shape[-1]
    for m in (2, 4, 8, 16, 32, 64):
        s = m // 2
        while s >= 1:
            if m < V:
                q = V // (2 * m)
                r = m // (2 * s)
                kr = k.reshape(q, 2, r, 2, s, 8, t)
                ir = ix.reshape(q, 2, r, 2, s, 8, t)
                a, b = kr[:, :, :, 0], kr[:, :, :, 1]
                ia_, ib_ = ir[:, :, :, 0], ir[:, :, :, 1]
                flip = (_sub_iota(a.shape, 1) == 1) ^ subflip_fn(
                    _sub_iota(a.shape, 4)
                )
                na, nia, nb, nib = _cex(a, ia_, b, ib_, flip)
                k = jnp.stack([na, nb], axis=3).reshape(V, 8, t)
                ix = jnp.stack([nia, nib], axis=3).reshape(V, 8, t)
            else:
                k, ix = _step_full(k, ix, s, subflip_fn)
            s //= 2
    return k, ix


def _merge64(k, ix, subflip_fn):
    s = V // 2
    while s >= 1:
        k, ix = _step_full(k, ix, s, subflip_fn)
        s //= 2
    return k, ix


def _sublane_shift(arr, d):
    return jnp.concatenate([arr[:, d:], arr[:, :d]], axis=1)


def _gate_kernel(x_ref, w_ref, idx_ref, scores_ref):
    x = x_ref[...]
    w = w_ref[...]
    logits = jax.lax.dot_general(
        w, x, (((1,), (1,)), ((), ())),
        preferred_element_type=jnp.float32,
    )
    t = logits.shape[-1]
    k = logits.reshape(V, 8, t)
    ix = _sub_iota((V, 8, t), 0) * 8 + _sub_iota((V, 8, t), 1)

    k, ix = _sort64(k, ix, lambda s: s % 2 == 1)

    merge_flips = [
        lambda s: (s % 4) == 2,
        lambda s: (s % 8) == 4,
        lambda s: s < 0,
    ]
    for rnd, d in enumerate((1, 2, 4)):
        pk = _sublane_shift(k, d)
        pi = _sublane_shift(ix, d)
        g = _gt(pk, pi, k, ix)
        k = jnp.where(g, pk, k)
        ix = jnp.where(g, pi, ix)
        k, ix = _merge64(k, ix, merge_flips[rnd])

    sub = _sub_iota((V, 8, t), 1)
    kv = jnp.max(jnp.where(sub == 0, k, -jnp.inf), axis=1)
    iv = jnp.max(jnp.where(sub == 0, ix, -1), axis=1)

    rank = _sub_iota((V, t), 0)
    e = jnp.where(rank < K, jnp.exp(kv - kv[0:1, :]), 0.0)
    ssum = jnp.sum(e, axis=0, keepdims=True)
    sc = e / ssum

    idx_ref[...] = iv
    scores_ref[...] = sc


@jax.jit
def kernel(x, W):
    n_tokens = x.shape[0]
    grid = (n_tokens // T_BLK,)
    idx_t, scores_t = pl.pallas_call(
        _gate_kernel,
        grid=grid,
        in_specs=[
            pl.BlockSpec((T_BLK, D_MODEL), lambda i: (i, 0)),
            pl.BlockSpec((NUM_PATHS, D_MODEL), lambda i: (0, 0)),
        ],
        out_specs=[
            pl.BlockSpec((V, T_BLK), lambda i: (0, i)),
            pl.BlockSpec((V, T_BLK), lambda i: (0, i)),
        ],
        out_shape=[
            jax.ShapeDtypeStruct((V, n_tokens), jnp.int32),
            jax.ShapeDtypeStruct((V, n_tokens), jnp.float32),
        ],
    )(x, W)
    return idx_t[:K].T, scores_t[:K].T


# speedup vs baseline: 5.8645x; 2.0500x over previous
import jax
import jax.numpy as jnp
from jax.experimental import pallas as pl

D_MODEL = 4096
NUM_PATHS = 512
K = 51
T_BLK = 256
V = 64


def _cex(ks, ix, i, j, flip):
    a, b = ks[i], ks[j]
    ia, ib = ix[i], ix[j]
    g = a < b
    if not flip:
        ks[i], ks[j] = jnp.where(g, b, a), jnp.where(g, a, b)
        ix[i], ix[j] = jnp.where(g, ib, ia), jnp.where(g, ia, ib)
    else:
        ks[i], ks[j] = jnp.where(g, a, b), jnp.where(g, b, a)
        ix[i], ix[j] = jnp.where(g, ia, ib), jnp.where(g, ib, ia)


def _sort64_desc(ks, ix):
    m = 2
    while m <= V:
        s = m // 2
        while s >= 1:
            for i in range(V):
                if i & s:
                    continue
                _cex(ks, ix, i, i | s, flip=bool(i & m) and m < V)
            s //= 2
        m *= 2


def _merge64_desc(ks, ix):
    s = V // 2
    while s >= 1:
        for i in range(V):
            if not i & s:
                _cex(ks, ix, i, i | s, flip=False)
        s //= 2


def _subrot(arr, d):
    return jnp.concatenate([arr[d:], arr[:d]], axis=0)


def _gate_kernel(x_ref, w_ref, idx_ref, scores_ref):
    x = x_ref[...]
    w = w_ref[...]
    logits = jax.lax.dot_general(
        w, x, (((1,), (1,)), ((), ())),
        preferred_element_type=jnp.float32,
    )
    t = logits.shape[-1]
    ks = [logits[8 * v: 8 * v + 8, :] for v in range(V)]
    sub = jax.lax.broadcasted_iota(jnp.int32, (8, t), 0)
    ix = [sub + 8 * v for v in range(V)]

    _sort64_desc(ks, ix)

    for d in (1, 2, 4):
        pks = [_subrot(ks[V - 1 - v], d) for v in range(V)]
        pix = [_subrot(ix[V - 1 - v], d) for v in range(V)]
        for v in range(V):
            g = pks[v] > ks[v]
            ks[v] = jnp.where(g, pks[v], ks[v])
            ix[v] = jnp.where(g, pix[v], ix[v])
        _merge64_desc(ks, ix)

    kv = jnp.concatenate([ks[r][0:1, :] for r in range(V)], axis=0)
    iv = jnp.concatenate([ix[r][0:1, :] for r in range(V)], axis=0)

    rank = jax.lax.broadcasted_iota(jnp.int32, (V, t), 0)
    e = jnp.where(rank < K, jnp.exp(kv - kv[0:1, :]), 0.0)
    ssum = jnp.sum(e, axis=0, keepdims=True)
    sc = e / ssum

    idx_ref[...] = iv
    scores_ref[...] = sc


@jax.jit
def kernel(x, W):
    n_tokens = x.shape[0]
    grid = (n_tokens // T_BLK,)
    idx_t, scores_t = pl.pallas_call(
        _gate_kernel,
        grid=grid,
        in_specs=[
            pl.BlockSpec((T_BLK, D_MODEL), lambda i: (i, 0)),
            pl.BlockSpec((NUM_PATHS, D_MODEL), lambda i: (0, 0)),
        ],
        out_specs=[
            pl.BlockSpec((V, T_BLK), lambda i: (0, i)),
            pl.BlockSpec((V, T_BLK), lambda i: (0, i)),
        ],
        out_shape=[
            jax.ShapeDtypeStruct((V, n_tokens), jnp.int32),
            jax.ShapeDtypeStruct((V, n_tokens), jnp.float32),
        ],
    )(x, W)
    return idx_t[:K].T, scores_t[:K].T


# speedup vs baseline: 5.8653x; 1.0001x over previous
import jax
import jax.numpy as jnp
from jax.experimental import pallas as pl
from jax.experimental.pallas import tpu as pltpu

D_MODEL = 4096
NUM_PATHS = 512
K = 51
T_BLK = 256
V = 64


def _cex(ks, ix, i, j, flip):
    a, b = ks[i], ks[j]
    ia, ib = ix[i], ix[j]
    g = a < b
    if not flip:
        ks[i], ks[j] = jnp.where(g, b, a), jnp.where(g, a, b)
        ix[i], ix[j] = jnp.where(g, ib, ia), jnp.where(g, ia, ib)
    else:
        ks[i], ks[j] = jnp.where(g, a, b), jnp.where(g, b, a)
        ix[i], ix[j] = jnp.where(g, ia, ib), jnp.where(g, ib, ia)


def _sort64_desc(ks, ix):
    m = 2
    while m <= V:
        s = m // 2
        while s >= 1:
            for i in range(V):
                if i & s:
                    continue
                _cex(ks, ix, i, i | s, flip=bool(i & m) and m < V)
            s //= 2
        m *= 2


def _merge64_desc(ks, ix):
    s = V // 2
    while s >= 1:
        for i in range(V):
            if not i & s:
                _cex(ks, ix, i, i | s, flip=False)
        s //= 2


def _subrot(arr, d):
    return pltpu.roll(arr, 8 - d, axis=0)


def _gate_kernel(x_ref, w_ref, idx_ref, scores_ref):
    x = x_ref[...]
    w = w_ref[...]
    logits = jax.lax.dot_general(
        w, x, (((1,), (1,)), ((), ())),
        preferred_element_type=jnp.float32,
    )
    t = logits.shape[-1]
    ks = [logits[8 * v: 8 * v + 8, :] for v in range(V)]
    sub = jax.lax.broadcasted_iota(jnp.int32, (8, t), 0)
    ix = [sub + 8 * v for v in range(V)]

    _sort64_desc(ks, ix)

    for d in (1, 2, 4):
        pks = [_subrot(ks[V - 1 - v], d) for v in range(V)]
        pix = [_subrot(ix[V - 1 - v], d) for v in range(V)]
        for v in range(V):
            g = pks[v] > ks[v]
            ks[v] = jnp.where(g, pks[v], ks[v])
            ix[v] = jnp.where(g, pix[v], ix[v])
        _merge64_desc(ks, ix)

    kv = jnp.concatenate([ks[r][0:1, :] for r in range(V)], axis=0)
    iv = jnp.concatenate([ix[r][0:1, :] for r in range(V)], axis=0)

    rank = jax.lax.broadcasted_iota(jnp.int32, (V, t), 0)
    e = jnp.where(rank < K, jnp.exp(kv - kv[0:1, :]), 0.0)
    ssum = jnp.sum(e, axis=0, keepdims=True)
    sc = e / ssum

    idx_ref[...] = iv
    scores_ref[...] = sc


@jax.jit
def kernel(x, W):
    n_tokens = x.shape[0]
    grid = (n_tokens // T_BLK,)
    idx_t, scores_t = pl.pallas_call(
        _gate_kernel,
        grid=grid,
        in_specs=[
            pl.BlockSpec((T_BLK, D_MODEL), lambda i: (i, 0)),
            pl.BlockSpec((NUM_PATHS, D_MODEL), lambda i: (0, 0)),
        ],
        out_specs=[
            pl.BlockSpec((V, T_BLK), lambda i: (0, i)),
            pl.BlockSpec((V, T_BLK), lambda i: (0, i)),
        ],
        out_shape=[
            jax.ShapeDtypeStruct((V, n_tokens), jnp.int32),
            jax.ShapeDtypeStruct((V, n_tokens), jnp.float32),
        ],
    )(x, W)
    return idx_t[:K].T, scores_t[:K].T
